# baseline (speedup 1.0000x reference)
import jax
import jax.numpy as jnp
from jax.experimental import pallas as pl

H_TILE = 8


def _pos_kernel(row_ref, col_ref, out_ref):
    row = row_ref[...]
    col = col_ref[...]
    out_ref[0] = row[:, None, :] + col[None, :, :]


def kernel(x, row_embed, col_embed):
    batch, height, width, _ = x.shape
    embed_dim = row_embed.shape[1]
    grid = (height // H_TILE, batch)
    return pl.pallas_call(
        _pos_kernel,
        grid=grid,
        in_specs=[
            pl.BlockSpec((H_TILE, embed_dim), lambda j, b: (j, 0)),
            pl.BlockSpec((width, embed_dim), lambda j, b: (0, 0)),
        ],
        out_specs=pl.BlockSpec(
            (1, H_TILE, width, embed_dim), lambda j, b: (b, j, 0, 0)
        ),
        out_shape=jax.ShapeDtypeStruct(
            (batch, height, width, embed_dim), row_embed.dtype
        ),
    )(row_embed, col_embed)


# speedup vs baseline: 2.0297x; 2.0297x over previous
import jax
import jax.numpy as jnp
from jax.experimental import pallas as pl

H_TILE = 64


def _pos_kernel(row_ref, col_ref, out_ref):
    row = row_ref[...]
    col = col_ref[...]
    out_ref[0] = row[:, None, :] + col[None, :, :]


def kernel(x, row_embed, col_embed):
    batch, height, width, _ = x.shape
    embed_dim = row_embed.shape[1]
    grid = (height // H_TILE, batch)
    return pl.pallas_call(
        _pos_kernel,
        grid=grid,
        in_specs=[
            pl.BlockSpec((H_TILE, embed_dim), lambda j, b: (j, 0)),
            pl.BlockSpec((width, embed_dim), lambda j, b: (0, 0)),
        ],
        out_specs=pl.BlockSpec(
            (1, H_TILE, width, embed_dim), lambda j, b: (b, j, 0, 0)
        ),
        out_shape=jax.ShapeDtypeStruct(
            (batch, height, width, embed_dim), row_embed.dtype
        ),
    )(row_embed, col_embed)
